# baseline (speedup 1.0000x reference)
import functools

import jax
import jax.numpy as jnp
from jax import lax
from jax.experimental import pallas as pl
from jax.experimental.pallas import tpu as pltpu
from jax.experimental.pallas import tpu_sc as plsc

N = 10000
E = 320000
D = 128
NP = 10240
CH = 128
NWORK = 32
NCH = 80
EPW = NCH * CH
EP = EPW * NWORK
RPT = NP // 16
GC = 8

_mesh = plsc.VectorSubcoreMesh(core_axis_name="c", subcore_axis_name="s")


def _deg_body(col_hbm, zero_hbm, out_hbm, ones_v, idx_v, deg_sh, sem):
    cid = lax.axis_index("c")
    sid = lax.axis_index("s")
    w = sid * 2 + cid
    for i in range(CH // 16):
        ones_v[pl.ds(i * 16, 16)] = jnp.full((16,), 1.0, jnp.float32)
    pltpu.sync_copy(col_hbm.at[w], idx_v)
    pltpu.sync_copy(zero_hbm.at[pl.ds(sid * RPT, RPT)],
                    deg_sh.at[pl.ds(sid * RPT, RPT)])
    plsc.subcore_barrier()

    def body(j, carry):
        pltpu.async_copy(ones_v, deg_sh.at[idx_v.at[j]], sem, add=True)
        return carry

    lax.fori_loop(0, NCH, body, 0)
    pltpu.make_async_copy(col_hbm.at[w], idx_v, sem).wait()
    plsc.subcore_barrier()
    pltpu.sync_copy(deg_sh.at[pl.ds(sid * RPT, RPT)],
                    out_hbm.at[cid, pl.ds(sid * RPT, RPT)])


_deg_kernel = functools.partial(
    pl.kernel,
    out_type=jax.ShapeDtypeStruct((2, NP), jnp.float32),
    mesh=_mesh,
    scratch_types=[
        pltpu.VMEM((CH,), jnp.float32),
        pltpu.VMEM((NCH, CH), jnp.int32),
        pltpu.VMEM_SHARED((NP,), jnp.float32),
        pltpu.SemaphoreType.DMA,
    ],
)(_deg_body)


def _agg_body(row_hbm, col_hbm, g_hbm, zero_hbm, out_hbm,
              ridx0, ridx1, cidx0, cidx1, rows_v, acc_sh,
              gsem, rsem0, rsem1, csem0, csem1):
    cid = lax.axis_index("c")
    sid = lax.axis_index("s")
    w = sid * 2 + cid
    pltpu.sync_copy(zero_hbm.at[pl.ds(sid * RPT, RPT)],
                    acc_sh.at[pl.ds(sid * RPT, RPT)])
    pltpu.async_copy(row_hbm.at[w, pl.ds(0, GC * CH)], ridx0, rsem0)
    pltpu.async_copy(row_hbm.at[w, pl.ds(GC * CH, GC * CH)], ridx1, rsem1)
    pltpu.async_copy(col_hbm.at[w, pl.ds(0, GC)], cidx0, csem0)
    pltpu.async_copy(col_hbm.at[w, pl.ds(GC, GC)], cidx1, csem1)
    plsc.subcore_barrier()

    def group(cbase, ridx, cidx, rsem, csem, ok_reload):
        pltpu.make_async_copy(row_hbm.at[w, pl.ds(0, GC * CH)],
                              ridx, rsem).wait()
        pltpu.make_async_copy(col_hbm.at[w, pl.ds(0, GC)],
                              cidx, csem).wait()
        for k in range(GC):
            pltpu.async_copy(g_hbm.at[ridx.at[pl.ds(k * CH, CH)]],
                             rows_v, gsem)
            pltpu.make_async_copy(g_hbm.at[ridx.at[pl.ds(k * CH, CH)]],
                                  rows_v, gsem).wait()
            pltpu.sync_copy(rows_v, acc_sh.at[cidx.at[k]], add=True)

        @pl.when(ok_reload)
        def _():
            pltpu.async_copy(
                row_hbm.at[w, pl.ds((cbase + 2 * GC) * CH, GC * CH)],
                ridx, rsem)
            pltpu.async_copy(col_hbm.at[w, pl.ds(cbase + 2 * GC, GC)],
                             cidx, csem)

    def body(q, carry):
        c = 2 * GC * q
        group(c, ridx0, cidx0, rsem0, csem0, c + 2 * GC < NCH)
        group(c + GC, ridx1, cidx1, rsem1, csem1, c + 3 * GC < NCH)
        return carry

    lax.fori_loop(0, NCH // (2 * GC), body, 0)
    plsc.subcore_barrier()
    pltpu.sync_copy(acc_sh.at[pl.ds(sid * RPT, RPT)],
                    out_hbm.at[cid, pl.ds(sid * RPT, RPT)])


_agg_kernel = functools.partial(
    pl.kernel,
    out_type=jax.ShapeDtypeStruct((2, NP, D), jnp.float32),
    mesh=_mesh,
    scratch_types=[
        pltpu.VMEM((GC * CH,), jnp.int32),
        pltpu.VMEM((GC * CH,), jnp.int32),
        pltpu.VMEM((GC, CH), jnp.int32),
        pltpu.VMEM((GC, CH), jnp.int32),
        pltpu.VMEM((CH, D), jnp.float32),
        pltpu.VMEM_SHARED((NP, D), jnp.float32),
        pltpu.SemaphoreType.DMA,
        pltpu.SemaphoreType.DMA,
        pltpu.SemaphoreType.DMA,
        pltpu.SemaphoreType.DMA,
        pltpu.SemaphoreType.DMA,
    ],
)(_agg_body)


BM = 1024


def _dis(degT_ref):
    deg = degT_ref[:, 0:1] + degT_ref[:, 1:2]
    return jnp.where(deg > 0, lax.rsqrt(deg), 0.0)


def _tc1_body(x_ref, w_ref, degT_ref, o_ref):
    dis = _dis(degT_ref)
    h = jnp.dot(x_ref[:, :], w_ref[:, :], preferred_element_type=jnp.float32)
    o_ref[:, :] = h * dis


def _tc2_body(acc_ref, degT_ref, b_ref, w_ref, o_ref):
    dis = _dis(degT_ref)
    a = (acc_ref[0] + acc_ref[1]) * dis + b_ref[:, :]
    h = jnp.maximum(a, 0.0)
    o_ref[:, :] = jnp.dot(h, w_ref[:, :],
                          preferred_element_type=jnp.float32) * dis


def _tc3_body(acc_ref, degT_ref, b_ref, o_ref):
    dis = _dis(degT_ref)
    o_ref[:, :] = (acc_ref[0] + acc_ref[1]) * dis + b_ref[:, :]


def _tc1(x_p, W1, degT):
    return pl.pallas_call(
        _tc1_body,
        out_shape=jax.ShapeDtypeStruct((NP, D), jnp.float32),
        grid=(NP // BM,),
        in_specs=[
            pl.BlockSpec((BM, D), lambda i: (i, 0)),
            pl.BlockSpec((D, D), lambda i: (0, 0)),
            pl.BlockSpec((BM, 2), lambda i: (i, 0)),
        ],
        out_specs=pl.BlockSpec((BM, D), lambda i: (i, 0)),
    )(x_p, W1, degT)


def _tc2(acc, degT, b1r, W2):
    return pl.pallas_call(
        _tc2_body,
        out_shape=jax.ShapeDtypeStruct((NP, D), jnp.float32),
        grid=(NP // BM,),
        in_specs=[
            pl.BlockSpec((2, BM, D), lambda i: (0, i, 0)),
            pl.BlockSpec((BM, 2), lambda i: (i, 0)),
            pl.BlockSpec((1, D), lambda i: (0, 0)),
            pl.BlockSpec((D, D), lambda i: (0, 0)),
        ],
        out_specs=pl.BlockSpec((BM, D), lambda i: (i, 0)),
    )(acc, degT, b1r, W2)


def _tc3(acc, degT, b2r):
    return pl.pallas_call(
        _tc3_body,
        out_shape=jax.ShapeDtypeStruct((NP, D), jnp.float32),
        grid=(NP // BM,),
        in_specs=[
            pl.BlockSpec((2, BM, D), lambda i: (0, i, 0)),
            pl.BlockSpec((BM, 2), lambda i: (i, 0)),
            pl.BlockSpec((1, D), lambda i: (0, 0)),
        ],
        out_specs=pl.BlockSpec((BM, D), lambda i: (i, 0)),
    )(acc, degT, b2r)


def kernel(x, edge_index, W1, b1, W2, b2):
    ei = edge_index.astype(jnp.int32)
    row_flat = jnp.concatenate([ei[0], jnp.zeros((EP - E,), jnp.int32)])
    col_flat = jnp.concatenate([ei[1], jnp.full((EP - E,), N, jnp.int32)])
    row = row_flat.reshape(NWORK, EPW)
    col_a = col_flat.reshape(NWORK, NCH, CH)
    col_d = col_flat.reshape(NWORK, NCH, CH)
    x_p = jnp.pad(x, ((0, NP - N), (0, 0)))
    z_deg = jnp.zeros((NP,), jnp.float32)
    z_nodes = jnp.zeros((NP, D), jnp.float32)
    b1r = b1.reshape(1, D)
    b2r = b2.reshape(1, D)

    deg2 = _deg_kernel(col_d, z_deg)
    degT = deg2.T
    g1 = _tc1(x_p, W1, degT)
    acc1 = _agg_kernel(row, col_a, g1, z_nodes)
    g2 = _tc2(acc1, degT, b1r, W2)
    acc2 = _agg_kernel(row, col_a, g2, z_nodes)
    out = _tc3(acc2, degT, b2r)
    return out[:N]


# speedup vs baseline: 1.4997x; 1.4997x over previous
import functools

import jax
import jax.numpy as jnp
from jax import lax
from jax.experimental import pallas as pl
from jax.experimental.pallas import tpu as pltpu
from jax.experimental.pallas import tpu_sc as plsc

N = 10000
E = 320000
D = 128
NP = 10240
CH = 128
NWORK = 32
NCH = 80
EPW = NCH * CH
EP = EPW * NWORK
RPT = NP // 16
GC = 8

_mesh = plsc.VectorSubcoreMesh(core_axis_name="c", subcore_axis_name="s")


def _deg_body(col_hbm, zero_hbm, out_hbm, ones_v, idx_v, deg_sh, sem):
    cid = lax.axis_index("c")
    sid = lax.axis_index("s")
    w = sid * 2 + cid
    for i in range(CH // 16):
        ones_v[pl.ds(i * 16, 16)] = jnp.full((16,), 1.0, jnp.float32)
    pltpu.sync_copy(col_hbm.at[w], idx_v)
    pltpu.sync_copy(zero_hbm.at[pl.ds(sid * RPT, RPT)],
                    deg_sh.at[pl.ds(sid * RPT, RPT)])
    plsc.subcore_barrier()

    def body(j, carry):
        pltpu.async_copy(ones_v, deg_sh.at[idx_v.at[j]], sem, add=True)
        return carry

    lax.fori_loop(0, NCH, body, 0)
    pltpu.make_async_copy(col_hbm.at[w], idx_v, sem).wait()
    plsc.subcore_barrier()
    pltpu.sync_copy(deg_sh.at[pl.ds(sid * RPT, RPT)],
                    out_hbm.at[cid, pl.ds(sid * RPT, RPT)])


_deg_kernel = functools.partial(
    pl.kernel,
    out_type=jax.ShapeDtypeStruct((2, NP), jnp.float32),
    mesh=_mesh,
    scratch_types=[
        pltpu.VMEM((CH,), jnp.float32),
        pltpu.VMEM((NCH, CH), jnp.int32),
        pltpu.VMEM_SHARED((NP,), jnp.float32),
        pltpu.SemaphoreType.DMA,
    ],
)(_deg_body)


NCH_C0 = 128
NCH_C1 = 32


def _agg_body(row_hbm, col_hbm, g_hbm, zero_hbm, out_hbm,
              ridx0, ridx1, cidx0, cidx1, rows_v, acc_sh,
              gsem, rsem0, rsem1, csem0, csem1):
    cid = lax.axis_index("c")
    sid = lax.axis_index("s")
    nch_w = jnp.where(cid == 0, NCH_C0, NCH_C1)
    cb0 = jnp.where(cid == 0, sid * NCH_C0, 16 * NCH_C0 + sid * NCH_C1)
    pltpu.sync_copy(zero_hbm.at[pl.ds(sid * RPT, RPT)],
                    acc_sh.at[pl.ds(sid * RPT, RPT)])
    pltpu.async_copy(row_hbm.at[pl.ds(cb0 * CH, GC * CH)], ridx0, rsem0)
    pltpu.async_copy(row_hbm.at[pl.ds((cb0 + GC) * CH, GC * CH)],
                     ridx1, rsem1)
    pltpu.async_copy(col_hbm.at[pl.ds(cb0, GC)], cidx0, csem0)
    pltpu.async_copy(col_hbm.at[pl.ds(cb0 + GC, GC)], cidx1, csem1)
    plsc.subcore_barrier()

    def group(cbase, ridx, cidx, rsem, csem, ok_reload):
        pltpu.make_async_copy(row_hbm.at[pl.ds(0, GC * CH)],
                              ridx, rsem).wait()
        pltpu.make_async_copy(col_hbm.at[pl.ds(0, GC)],
                              cidx, csem).wait()
        for k in range(GC):
            pltpu.async_copy(g_hbm.at[ridx.at[pl.ds(k * CH, CH)]],
                             rows_v, gsem)
            pltpu.make_async_copy(g_hbm.at[ridx.at[pl.ds(k * CH, CH)]],
                                  rows_v, gsem).wait()
            pltpu.sync_copy(rows_v, acc_sh.at[cidx.at[k]], add=True)

        @pl.when(ok_reload)
        def _():
            pltpu.async_copy(
                row_hbm.at[pl.ds((cbase + 2 * GC) * CH, GC * CH)],
                ridx, rsem)
            pltpu.async_copy(col_hbm.at[pl.ds(cbase + 2 * GC, GC)],
                             cidx, csem)

    def body(q, carry):
        c = cb0 + 2 * GC * q
        group(c, ridx0, cidx0, rsem0, csem0, c + 2 * GC < cb0 + nch_w)
        group(c + GC, ridx1, cidx1, rsem1, csem1, c + 3 * GC < cb0 + nch_w)
        return carry

    lax.fori_loop(0, nch_w // (2 * GC), body, 0)
    plsc.subcore_barrier()
    pltpu.sync_copy(acc_sh.at[pl.ds(sid * RPT, RPT)],
                    out_hbm.at[cid, pl.ds(sid * RPT, RPT)])


_agg_kernel = functools.partial(
    pl.kernel,
    out_type=jax.ShapeDtypeStruct((2, NP, D), jnp.float32),
    mesh=_mesh,
    scratch_types=[
        pltpu.VMEM((GC * CH,), jnp.int32),
        pltpu.VMEM((GC * CH,), jnp.int32),
        pltpu.VMEM((GC, CH), jnp.int32),
        pltpu.VMEM((GC, CH), jnp.int32),
        pltpu.VMEM((CH, D), jnp.float32),
        pltpu.VMEM_SHARED((NP, D), jnp.float32),
        pltpu.SemaphoreType.DMA,
        pltpu.SemaphoreType.DMA,
        pltpu.SemaphoreType.DMA,
        pltpu.SemaphoreType.DMA,
        pltpu.SemaphoreType.DMA,
    ],
)(_agg_body)


BM = 1024


def _dis(degT_ref):
    deg = degT_ref[:, 0:1] + degT_ref[:, 1:2]
    return jnp.where(deg > 0, lax.rsqrt(deg), 0.0)


def _tc1_body(x_ref, w_ref, degT_ref, o_ref):
    dis = _dis(degT_ref)
    h = jnp.dot(x_ref[:, :], w_ref[:, :], preferred_element_type=jnp.float32)
    o_ref[:, :] = h * dis


def _tc2_body(acc_ref, degT_ref, b_ref, w_ref, o_ref):
    dis = _dis(degT_ref)
    a = (acc_ref[0] + acc_ref[1]) * dis + b_ref[:, :]
    h = jnp.maximum(a, 0.0)
    o_ref[:, :] = jnp.dot(h, w_ref[:, :],
                          preferred_element_type=jnp.float32) * dis


def _tc3_body(acc_ref, degT_ref, b_ref, o_ref):
    dis = _dis(degT_ref)
    o_ref[:, :] = (acc_ref[0] + acc_ref[1]) * dis + b_ref[:, :]


def _tc1(x_p, W1, degT):
    return pl.pallas_call(
        _tc1_body,
        out_shape=jax.ShapeDtypeStruct((NP, D), jnp.float32),
        grid=(NP // BM,),
        in_specs=[
            pl.BlockSpec((BM, D), lambda i: (i, 0)),
            pl.BlockSpec((D, D), lambda i: (0, 0)),
            pl.BlockSpec((BM, 2), lambda i: (i, 0)),
        ],
        out_specs=pl.BlockSpec((BM, D), lambda i: (i, 0)),
    )(x_p, W1, degT)


def _tc2(acc, degT, b1r, W2):
    return pl.pallas_call(
        _tc2_body,
        out_shape=jax.ShapeDtypeStruct((NP, D), jnp.float32),
        grid=(NP // BM,),
        in_specs=[
            pl.BlockSpec((2, BM, D), lambda i: (0, i, 0)),
            pl.BlockSpec((BM, 2), lambda i: (i, 0)),
            pl.BlockSpec((1, D), lambda i: (0, 0)),
            pl.BlockSpec((D, D), lambda i: (0, 0)),
        ],
        out_specs=pl.BlockSpec((BM, D), lambda i: (i, 0)),
    )(acc, degT, b1r, W2)


def _tc3(acc, degT, b2r):
    return pl.pallas_call(
        _tc3_body,
        out_shape=jax.ShapeDtypeStruct((NP, D), jnp.float32),
        grid=(NP // BM,),
        in_specs=[
            pl.BlockSpec((2, BM, D), lambda i: (0, i, 0)),
            pl.BlockSpec((BM, 2), lambda i: (i, 0)),
            pl.BlockSpec((1, D), lambda i: (0, 0)),
        ],
        out_specs=pl.BlockSpec((BM, D), lambda i: (i, 0)),
    )(acc, degT, b2r)


def kernel(x, edge_index, W1, b1, W2, b2):
    ei = edge_index.astype(jnp.int32)
    row_flat = jnp.concatenate([ei[0], jnp.zeros((EP - E,), jnp.int32)])
    col_flat = jnp.concatenate([ei[1], jnp.full((EP - E,), N, jnp.int32)])
    row = row_flat
    col_a = col_flat.reshape(EP // CH, CH)
    col_d = col_flat.reshape(NWORK, NCH, CH)
    x_p = jnp.pad(x, ((0, NP - N), (0, 0)))
    z_deg = jnp.zeros((NP,), jnp.float32)
    z_nodes = jnp.zeros((NP, D), jnp.float32)
    b1r = b1.reshape(1, D)
    b2r = b2.reshape(1, D)

    deg2 = _deg_kernel(col_d, z_deg)
    degT = deg2.T
    g1 = _tc1(x_p, W1, degT)
    acc1 = _agg_kernel(row, col_a, g1, z_nodes)
    g2 = _tc2(acc1, degT, b1r, W2)
    acc2 = _agg_kernel(row, col_a, g2, z_nodes)
    out = _tc3(acc2, degT, b2r)
    return out[:N]


# speedup vs baseline: 1.5229x; 1.0154x over previous
import functools

import jax
import jax.numpy as jnp
from jax import lax
from jax.experimental import pallas as pl
from jax.experimental.pallas import tpu as pltpu
from jax.experimental.pallas import tpu_sc as plsc

N = 10000
E = 320000
D = 128
NP = 10240
CH = 128
NWORK = 32
NCH = 80
EPW = NCH * CH
EP = EPW * NWORK
RPT = NP // 16
GC = 8

_mesh = plsc.VectorSubcoreMesh(core_axis_name="c", subcore_axis_name="s")


def _deg_body(col_hbm, zero_hbm, out_hbm, ones_v, idx_v, deg_sh, sem):
    cid = lax.axis_index("c")
    sid = lax.axis_index("s")
    w = sid * 2 + cid
    for i in range(CH // 16):
        ones_v[pl.ds(i * 16, 16)] = jnp.full((16,), 1.0, jnp.float32)
    pltpu.sync_copy(col_hbm.at[w], idx_v)
    pltpu.sync_copy(zero_hbm.at[pl.ds(sid * RPT, RPT)],
                    deg_sh.at[pl.ds(sid * RPT, RPT)])
    plsc.subcore_barrier()

    def body(j, carry):
        pltpu.async_copy(ones_v, deg_sh.at[idx_v.at[j]], sem, add=True)
        return carry

    lax.fori_loop(0, NCH, body, 0)
    pltpu.make_async_copy(col_hbm.at[w], idx_v, sem).wait()
    plsc.subcore_barrier()
    pltpu.sync_copy(deg_sh.at[pl.ds(sid * RPT, RPT)],
                    out_hbm.at[cid, pl.ds(sid * RPT, RPT)])


_deg_kernel = functools.partial(
    pl.kernel,
    out_type=jax.ShapeDtypeStruct((2, NP), jnp.float32),
    mesh=_mesh,
    scratch_types=[
        pltpu.VMEM((CH,), jnp.float32),
        pltpu.VMEM((NCH, CH), jnp.int32),
        pltpu.VMEM_SHARED((NP,), jnp.float32),
        pltpu.SemaphoreType.DMA,
    ],
)(_deg_body)


NCH_C0 = 144
NCH_C1 = 16


def _agg_body(row_hbm, col_hbm, g_hbm, zero_hbm, out_hbm,
              ridx0, ridx1, cidx0, cidx1, rows_v, acc_sh,
              gsem, rsem0, rsem1, csem0, csem1):
    cid = lax.axis_index("c")
    sid = lax.axis_index("s")
    nch_w = jnp.where(cid == 0, NCH_C0, NCH_C1)
    cb0 = jnp.where(cid == 0, sid * NCH_C0, 16 * NCH_C0 + sid * NCH_C1)
    pltpu.sync_copy(zero_hbm.at[pl.ds(sid * RPT, RPT)],
                    acc_sh.at[pl.ds(sid * RPT, RPT)])
    pltpu.async_copy(row_hbm.at[pl.ds(cb0 * CH, GC * CH)], ridx0, rsem0)
    pltpu.async_copy(row_hbm.at[pl.ds((cb0 + GC) * CH, GC * CH)],
                     ridx1, rsem1)
    pltpu.async_copy(col_hbm.at[pl.ds(cb0, GC)], cidx0, csem0)
    pltpu.async_copy(col_hbm.at[pl.ds(cb0 + GC, GC)], cidx1, csem1)
    plsc.subcore_barrier()

    def group(cbase, ridx, cidx, rsem, csem, ok_reload):
        pltpu.make_async_copy(row_hbm.at[pl.ds(0, GC * CH)],
                              ridx, rsem).wait()
        pltpu.make_async_copy(col_hbm.at[pl.ds(0, GC)],
                              cidx, csem).wait()
        for k in range(GC):
            pltpu.async_copy(g_hbm.at[ridx.at[pl.ds(k * CH, CH)]],
                             rows_v, gsem)
            pltpu.make_async_copy(g_hbm.at[ridx.at[pl.ds(k * CH, CH)]],
                                  rows_v, gsem).wait()
            pltpu.sync_copy(rows_v, acc_sh.at[cidx.at[k]], add=True)

        @pl.when(ok_reload)
        def _():
            pltpu.async_copy(
                row_hbm.at[pl.ds((cbase + 2 * GC) * CH, GC * CH)],
                ridx, rsem)
            pltpu.async_copy(col_hbm.at[pl.ds(cbase + 2 * GC, GC)],
                             cidx, csem)

    def body(q, carry):
        c = cb0 + 2 * GC * q
        group(c, ridx0, cidx0, rsem0, csem0, c + 2 * GC < cb0 + nch_w)
        group(c + GC, ridx1, cidx1, rsem1, csem1, c + 3 * GC < cb0 + nch_w)
        return carry

    lax.fori_loop(0, nch_w // (2 * GC), body, 0)
    plsc.subcore_barrier()
    pltpu.sync_copy(acc_sh.at[pl.ds(sid * RPT, RPT)],
                    out_hbm.at[cid, pl.ds(sid * RPT, RPT)])


_agg_kernel = functools.partial(
    pl.kernel,
    out_type=jax.ShapeDtypeStruct((2, NP, D), jnp.float32),
    mesh=_mesh,
    scratch_types=[
        pltpu.VMEM((GC * CH,), jnp.int32),
        pltpu.VMEM((GC * CH,), jnp.int32),
        pltpu.VMEM((GC, CH), jnp.int32),
        pltpu.VMEM((GC, CH), jnp.int32),
        pltpu.VMEM((CH, D), jnp.float32),
        pltpu.VMEM_SHARED((NP, D), jnp.float32),
        pltpu.SemaphoreType.DMA,
        pltpu.SemaphoreType.DMA,
        pltpu.SemaphoreType.DMA,
        pltpu.SemaphoreType.DMA,
        pltpu.SemaphoreType.DMA,
    ],
)(_agg_body)


BM = 1024


def _dis(degT_ref):
    deg = degT_ref[:, 0:1] + degT_ref[:, 1:2]
    return jnp.where(deg > 0, lax.rsqrt(deg), 0.0)


def _tc1_body(x_ref, w_ref, degT_ref, o_ref):
    dis = _dis(degT_ref)
    h = jnp.dot(x_ref[:, :], w_ref[:, :], preferred_element_type=jnp.float32)
    o_ref[:, :] = h * dis


def _tc2_body(acc_ref, degT_ref, b_ref, w_ref, o_ref):
    dis = _dis(degT_ref)
    a = (acc_ref[0] + acc_ref[1]) * dis + b_ref[:, :]
    h = jnp.maximum(a, 0.0)
    o_ref[:, :] = jnp.dot(h, w_ref[:, :],
                          preferred_element_type=jnp.float32) * dis


def _tc3_body(acc_ref, degT_ref, b_ref, o_ref):
    dis = _dis(degT_ref)
    o_ref[:, :] = (acc_ref[0] + acc_ref[1]) * dis + b_ref[:, :]


def _tc1(x_p, W1, degT):
    return pl.pallas_call(
        _tc1_body,
        out_shape=jax.ShapeDtypeStruct((NP, D), jnp.float32),
        grid=(NP // BM,),
        in_specs=[
            pl.BlockSpec((BM, D), lambda i: (i, 0)),
            pl.BlockSpec((D, D), lambda i: (0, 0)),
            pl.BlockSpec((BM, 2), lambda i: (i, 0)),
        ],
        out_specs=pl.BlockSpec((BM, D), lambda i: (i, 0)),
    )(x_p, W1, degT)


def _tc2(acc, degT, b1r, W2):
    return pl.pallas_call(
        _tc2_body,
        out_shape=jax.ShapeDtypeStruct((NP, D), jnp.float32),
        grid=(NP // BM,),
        in_specs=[
            pl.BlockSpec((2, BM, D), lambda i: (0, i, 0)),
            pl.BlockSpec((BM, 2), lambda i: (i, 0)),
            pl.BlockSpec((1, D), lambda i: (0, 0)),
            pl.BlockSpec((D, D), lambda i: (0, 0)),
        ],
        out_specs=pl.BlockSpec((BM, D), lambda i: (i, 0)),
    )(acc, degT, b1r, W2)


def _tc3(acc, degT, b2r):
    return pl.pallas_call(
        _tc3_body,
        out_shape=jax.ShapeDtypeStruct((NP, D), jnp.float32),
        grid=(NP // BM,),
        in_specs=[
            pl.BlockSpec((2, BM, D), lambda i: (0, i, 0)),
            pl.BlockSpec((BM, 2), lambda i: (i, 0)),
            pl.BlockSpec((1, D), lambda i: (0, 0)),
        ],
        out_specs=pl.BlockSpec((BM, D), lambda i: (i, 0)),
    )(acc, degT, b2r)


def kernel(x, edge_index, W1, b1, W2, b2):
    ei = edge_index.astype(jnp.int32)
    row_flat = jnp.concatenate([ei[0], jnp.zeros((EP - E,), jnp.int32)])
    col_flat = jnp.concatenate([ei[1], jnp.full((EP - E,), N, jnp.int32)])
    row = row_flat
    col_a = col_flat.reshape(EP // CH, CH)
    col_d = col_flat.reshape(NWORK, NCH, CH)
    x_p = jnp.pad(x, ((0, NP - N), (0, 0)))
    z_deg = jnp.zeros((NP,), jnp.float32)
    z_nodes = jnp.zeros((NP, D), jnp.float32)
    b1r = b1.reshape(1, D)
    b2r = b2.reshape(1, D)

    deg2 = _deg_kernel(col_d, z_deg)
    degT = deg2.T
    g1 = _tc1(x_p, W1, degT)
    acc1 = _agg_kernel(row, col_a, g1, z_nodes)
    g2 = _tc2(acc1, degT, b1r, W2)
    acc2 = _agg_kernel(row, col_a, g2, z_nodes)
    out = _tc3(acc2, degT, b2r)
    return out[:N]
